# baseline (speedup 1.0000x reference)
import jax
import jax.numpy as jnp
from jax import lax
from jax.experimental import pallas as pl
from jax.experimental.pallas import tpu as pltpu
from jax.experimental.pallas import tpu_sc as plsc

N = 10000
E = 160000
D_IN = 256
D_H = 512
D_OUT = 256

NCORE = 2
NT = 16
C = 128
WIN = 128
NW = 79
EPAD = NT * NW * WIN
NPAD = NT * 640
STRIPE = NPAD // NT


def _make_sc_agg(K):
  mesh = plsc.VectorSubcoreMesh(
      core_axis_name="c", subcore_axis_name="s", num_cores=NCORE,
      num_subcores=NT)

  def body(table_hbm, src_hbm, dst_hbm, out_hbm,
           src_v, dst_v, rows_v, acc_sh):
    c = lax.axis_index("c")
    s = lax.axis_index("s")

    pltpu.sync_copy(src_hbm.at[s], src_v)
    pltpu.sync_copy(dst_hbm.at[s], dst_v)

    for k in range(K):
      @pl.when(c == (k % NCORE))
      def _():
        @pl.loop(0, WIN)
        def _(i):
          @pl.loop(0, C, step=16)
          def _(j):
            rows_v[i, pl.ds(j, 16)] = jnp.zeros((16,), jnp.float32)

        for r in range(STRIPE // WIN):
          pltpu.sync_copy(rows_v, acc_sh.at[pl.ds(s * STRIPE + r * WIN, WIN)])
        plsc.subcore_barrier()

        @pl.loop(0, NW)
        def _(w):
          pltpu.sync_copy(table_hbm.at[k].at[src_v.at[w]], rows_v)
          pltpu.sync_copy(rows_v, acc_sh.at[dst_v.at[w]], add=True)

        plsc.subcore_barrier()
        pltpu.sync_copy(acc_sh.at[pl.ds(s * STRIPE, STRIPE)],
                        out_hbm.at[k].at[pl.ds(s * STRIPE, STRIPE)])
        plsc.subcore_barrier()

  return pl.kernel(
      body,
      out_type=jax.ShapeDtypeStruct((K, NPAD, C), jnp.float32),
      mesh=mesh,
      scratch_types=[
          pltpu.VMEM((NW, WIN), jnp.int32),
          pltpu.VMEM((NW, WIN), jnp.int32),
          pltpu.VMEM((WIN, C), jnp.float32),
          pltpu.VMEM_SHARED((NPAD, C), jnp.float32),
      ])


_sc_agg_3 = _make_sc_agg(K=3)
_sc_agg_4 = _make_sc_agg(K=4)
_sc_agg_2 = _make_sc_agg(K=2)


RB = 1000
GRID = N // RB


def _inv_deg(deg_blk):
  return 1.0 / jnp.maximum(deg_blk[0, :, 0:1], 1.0)


def _self_body(h_ref, w_ref, b_ref, o_ref):
  o_ref[...] = (jnp.dot(h_ref[...], w_ref[...],
                        preferred_element_type=jnp.float32) + b_ref[...])


def _combine0_body(s_ref, agg_ref, wn_ref, h_ref, hc_ref):
  inv = 1.0 / jnp.maximum(agg_ref[2, :, 0:1], 1.0)
  neigh = jnp.concatenate([agg_ref[0], agg_ref[1]], axis=1) * inv
  h = s_ref[...] + jnp.dot(neigh, wn_ref[...],
                           preferred_element_type=jnp.float32)
  h = jnp.maximum(h, 0.0)
  h_ref[...] = h
  for kk in range(4):
    hc_ref[kk] = h[:, kk * C:(kk + 1) * C]


def _combine1_body(s_ref, agg_ref, deg_ref, wn_ref, wn2_ref, h_ref, pc_ref):
  inv = _inv_deg(deg_ref[...])
  neigh = jnp.concatenate([agg_ref[kk] for kk in range(4)], axis=1) * inv
  h = s_ref[...] + jnp.dot(neigh, wn_ref[...],
                           preferred_element_type=jnp.float32)
  h = jnp.maximum(h, 0.0)
  h_ref[...] = h
  p = jnp.dot(h, wn2_ref[...], preferred_element_type=jnp.float32)
  for kk in range(2):
    pc_ref[kk] = p[:, kk * C:(kk + 1) * C]


def _combine2_body(s_ref, agg_ref, deg_ref, out_ref):
  inv = _inv_deg(deg_ref[...])
  neigh = jnp.concatenate([agg_ref[0], agg_ref[1]], axis=1) * inv
  out_ref[...] = s_ref[...] + neigh


def _row_spec(d):
  return pl.BlockSpec((RB, d), lambda i: (i, 0))


def _deg_spec():
  return pl.BlockSpec((1, RB, C), lambda i: (2, i, 0))


def _chunk_spec(k):
  return pl.BlockSpec((k, RB, C), lambda i: (0, i, 0))


def _full_spec(shape):
  nd = len(shape)
  return pl.BlockSpec(shape, lambda i: (0,) * nd)


def _make_self(din, dout):
  return pl.pallas_call(
      _self_body,
      grid=(GRID,),
      in_specs=[_row_spec(din), _full_spec((din, dout)),
                _full_spec((1, dout))],
      out_specs=_row_spec(dout),
      out_shape=jax.ShapeDtypeStruct((N, dout), jnp.float32),
  )


_self_0 = _make_self(D_IN, D_H)
_self_1 = _make_self(D_H, D_H)
_self_2 = _make_self(D_H, D_OUT)

_combine0 = pl.pallas_call(
    _combine0_body,
    grid=(GRID,),
    in_specs=[_row_spec(D_H), _chunk_spec(3), _full_spec((D_IN, D_H))],
    out_specs=[_row_spec(D_H), _chunk_spec(4)],
    out_shape=[jax.ShapeDtypeStruct((N, D_H), jnp.float32),
               jax.ShapeDtypeStruct((4, N, C), jnp.float32)],
)

_combine1 = pl.pallas_call(
    _combine1_body,
    grid=(GRID,),
    in_specs=[_row_spec(D_H), _chunk_spec(4), _deg_spec(),
              _full_spec((D_H, D_H)), _full_spec((D_H, D_OUT))],
    out_specs=[_row_spec(D_H), _chunk_spec(2)],
    out_shape=[jax.ShapeDtypeStruct((N, D_H), jnp.float32),
               jax.ShapeDtypeStruct((2, N, C), jnp.float32)],
)

_combine2 = pl.pallas_call(
    _combine2_body,
    grid=(GRID,),
    in_specs=[_row_spec(D_OUT), _chunk_spec(2), _deg_spec()],
    out_specs=_row_spec(D_OUT),
    out_shape=jax.ShapeDtypeStruct((N, D_OUT), jnp.float32),
)


def kernel(x, edge_index, W_self_0, W_neigh_0, b_0, W_self_1, W_neigh_1, b_1,
           W_self_2, W_neigh_2, b_2):
  src = edge_index[0]
  dst = edge_index[1]
  npad_e = EPAD - E
  pad_src = jnp.zeros((npad_e,), jnp.int32)
  pad_dst = N + (jnp.arange(npad_e, dtype=jnp.int32) % (NPAD - N))
  src_p = jnp.concatenate([src, pad_src]).reshape(NT, NW, WIN)
  dst_p = jnp.concatenate([dst, pad_dst]).reshape(NT, NW, WIN)

  x_c = x.reshape(N, 2, C).transpose(1, 0, 2)

  table0 = jnp.concatenate([x_c, jnp.ones((1, N, C), jnp.float32)], axis=0)
  agg0 = _sc_agg_3(table0, src_p, dst_p)
  s0 = _self_0(x, W_self_0, b_0.reshape(1, D_H))
  h0, h0c = _combine0(s0, agg0, W_neigh_0)
  agg1 = _sc_agg_4(h0c, src_p, dst_p)
  s1 = _self_1(h0, W_self_1, b_1.reshape(1, D_H))
  h1, pc = _combine1(s1, agg1, agg0, W_neigh_1, W_neigh_2)
  agg2 = _sc_agg_2(pc, src_p, dst_p)
  s2 = _self_2(h1, W_self_2, b_2.reshape(1, D_OUT))
  out = _combine2(s2, agg2, agg0)
  return out


# speedup vs baseline: 1.1417x; 1.1417x over previous
import jax
import jax.numpy as jnp
from jax import lax
from jax.experimental import pallas as pl
from jax.experimental.pallas import tpu as pltpu
from jax.experimental.pallas import tpu_sc as plsc

N = 10000
E = 160000
D_IN = 256
D_H = 512
D_OUT = 256

NCORE = 2
NT = 16
C = 128
WIN = 128
NW = 79
EPAD = NT * NW * WIN
NPAD = NT * 640
STRIPE = NPAD // NT


def _make_sc_agg(K, with_ones=False):
  KT = K + (1 if with_ones else 0)
  mesh = plsc.VectorSubcoreMesh(
      core_axis_name="c", subcore_axis_name="s", num_cores=NCORE,
      num_subcores=NT)

  def body(table_hbm, src_hbm, dst_hbm, out_hbm,
           src_v, dst_v, rows_v, acc_sh):
    c = lax.axis_index("c")
    s = lax.axis_index("s")

    pltpu.sync_copy(src_hbm.at[s], src_v)
    pltpu.sync_copy(dst_hbm.at[s], dst_v)

    def fill(val):
      @pl.loop(0, WIN)
      def _(i):
        @pl.loop(0, C, step=16)
        def _(j):
          rows_v[i, pl.ds(j, 16)] = jnp.full((16,), val, jnp.float32)

    for k in range(KT):
      ones_pass = with_ones and k == K

      @pl.when(c == (k % NCORE))
      def _():
        fill(0.0)
        for r in range(STRIPE // WIN):
          pltpu.sync_copy(rows_v, acc_sh.at[pl.ds(s * STRIPE + r * WIN, WIN)])
        plsc.subcore_barrier()

        if ones_pass:
          fill(1.0)

          @pl.loop(0, NW)
          def _(w):
            pltpu.sync_copy(rows_v, acc_sh.at[dst_v.at[w]], add=True)
        else:
          @pl.loop(0, NW)
          def _(w):
            pltpu.sync_copy(table_hbm.at[k].at[src_v.at[w]], rows_v)
            pltpu.sync_copy(rows_v, acc_sh.at[dst_v.at[w]], add=True)

        plsc.subcore_barrier()
        pltpu.sync_copy(acc_sh.at[pl.ds(s * STRIPE, STRIPE)],
                        out_hbm.at[k].at[pl.ds(s * STRIPE, STRIPE)])
        plsc.subcore_barrier()

  return pl.kernel(
      body,
      out_type=jax.ShapeDtypeStruct((KT, NPAD, C), jnp.float32),
      mesh=mesh,
      scratch_types=[
          pltpu.VMEM((NW, WIN), jnp.int32),
          pltpu.VMEM((NW, WIN), jnp.int32),
          pltpu.VMEM((WIN, C), jnp.float32),
          pltpu.VMEM_SHARED((NPAD, C), jnp.float32),
      ])


_sc_agg_3 = _make_sc_agg(K=2, with_ones=True)
_sc_agg_4 = _make_sc_agg(K=4)
_sc_agg_2 = _make_sc_agg(K=2)


RB = 1000
GRID = N // RB


def _inv_deg(deg_blk):
  return 1.0 / jnp.maximum(deg_blk[0, :, 0:1], 1.0)


def _combine0_body(x_ref, agg_ref, ws_ref, wn_ref, b_ref,
                   h_ref, hc_ref):
  inv = 1.0 / jnp.maximum(agg_ref[2, :, 0:1], 1.0)
  neigh = jnp.concatenate([agg_ref[0], agg_ref[1]], axis=1) * inv
  h = (jnp.dot(x_ref[...], ws_ref[...], preferred_element_type=jnp.float32)
       + jnp.dot(neigh, wn_ref[...], preferred_element_type=jnp.float32)
       + b_ref[...])
  h = jnp.maximum(h, 0.0)
  h_ref[...] = h
  for kk in range(4):
    hc_ref[kk] = h[:, kk * C:(kk + 1) * C]


def _combine1_body(h0_ref, agg_ref, deg_ref, ws_ref, wn_ref, b_ref, wn2_ref,
                   h_ref, pc_ref):
  inv = _inv_deg(deg_ref[...])
  neigh = jnp.concatenate([agg_ref[kk] for kk in range(4)], axis=1) * inv
  h = (jnp.dot(h0_ref[...], ws_ref[...], preferred_element_type=jnp.float32)
       + jnp.dot(neigh, wn_ref[...], preferred_element_type=jnp.float32)
       + b_ref[...])
  h = jnp.maximum(h, 0.0)
  h_ref[...] = h
  p = jnp.dot(h, wn2_ref[...], preferred_element_type=jnp.float32)
  for kk in range(2):
    pc_ref[kk] = p[:, kk * C:(kk + 1) * C]


def _combine2_body(h1_ref, agg_ref, deg_ref, ws_ref, b_ref, out_ref):
  inv = _inv_deg(deg_ref[...])
  neigh = jnp.concatenate([agg_ref[0], agg_ref[1]], axis=1) * inv
  out_ref[...] = (
      jnp.dot(h1_ref[...], ws_ref[...], preferred_element_type=jnp.float32)
      + neigh + b_ref[...])


def _row_spec(d):
  return pl.BlockSpec((RB, d), lambda i: (i, 0))


def _deg_spec():
  return pl.BlockSpec((1, RB, C), lambda i: (2, i, 0))


def _chunk_spec(k):
  return pl.BlockSpec((k, RB, C), lambda i: (0, i, 0))


def _full_spec(shape):
  nd = len(shape)
  return pl.BlockSpec(shape, lambda i: (0,) * nd)


_combine0 = pl.pallas_call(
    _combine0_body,
    grid=(GRID,),
    in_specs=[_row_spec(D_IN), _chunk_spec(3),
              _full_spec((D_IN, D_H)), _full_spec((D_IN, D_H)),
              _full_spec((1, D_H))],
    out_specs=[_row_spec(D_H), _chunk_spec(4)],
    out_shape=[jax.ShapeDtypeStruct((N, D_H), jnp.float32),
               jax.ShapeDtypeStruct((4, N, C), jnp.float32)],
)

_combine1 = pl.pallas_call(
    _combine1_body,
    grid=(GRID,),
    in_specs=[_row_spec(D_H), _chunk_spec(4), _deg_spec(),
              _full_spec((D_H, D_H)), _full_spec((D_H, D_H)),
              _full_spec((1, D_H)), _full_spec((D_H, D_OUT))],
    out_specs=[_row_spec(D_H), _chunk_spec(2)],
    out_shape=[jax.ShapeDtypeStruct((N, D_H), jnp.float32),
               jax.ShapeDtypeStruct((2, N, C), jnp.float32)],
)

_combine2 = pl.pallas_call(
    _combine2_body,
    grid=(GRID,),
    in_specs=[_row_spec(D_H), _chunk_spec(2), _deg_spec(),
              _full_spec((D_H, D_OUT)), _full_spec((1, D_OUT))],
    out_specs=_row_spec(D_OUT),
    out_shape=jax.ShapeDtypeStruct((N, D_OUT), jnp.float32),
)


def kernel(x, edge_index, W_self_0, W_neigh_0, b_0, W_self_1, W_neigh_1, b_1,
           W_self_2, W_neigh_2, b_2):
  src = edge_index[0]
  dst = edge_index[1]
  npad_e = EPAD - E
  pad_src = jnp.zeros((npad_e,), jnp.int32)
  pad_dst = N + (jnp.arange(npad_e, dtype=jnp.int32) % (NPAD - N))
  src_p = jnp.concatenate([src, pad_src]).reshape(NT, NW, WIN)
  dst_p = jnp.concatenate([dst, pad_dst]).reshape(NT, NW, WIN)

  x_c = x.reshape(N, 2, C).transpose(1, 0, 2)

  agg0 = _sc_agg_3(x_c, src_p, dst_p)
  h0, h0c = _combine0(x, agg0, W_self_0, W_neigh_0, b_0.reshape(1, D_H))
  agg1 = _sc_agg_4(h0c, src_p, dst_p)
  h1, pc = _combine1(h0, agg1, agg0, W_self_1, W_neigh_1,
                     b_1.reshape(1, D_H), W_neigh_2)
  agg2 = _sc_agg_2(pc, src_p, dst_p)
  out = _combine2(h1, agg2, agg0, W_self_2, b_2.reshape(1, D_OUT))
  return out
